# baseline (speedup 1.0000x reference)
import jax
import jax.numpy as jnp
from jax.experimental import pallas as pl
from jax.experimental.pallas import tpu as pltpu


_N = 4096
_F = 128
_BMS = 128
_NBS = _N // _BMS
_BMR = 512
_NBR = _N // _BMR
_BMG = 256
_NBG = _N // _BMG


def _model_kernel(adj_ref, x_ref, w1_ref, w2_ref, w3_ref, w4_ref, w5_ref,
                  w6_ref, zig_ref, g1_ref, zhat_ref, g2_ref,
                  adj16_ref, feat_ref, sup_ref, zb_ref, slot0_ref, slot1_ref,
                  sem_ref):
    i = pl.program_id(0)

    @pl.when(i == 0)
    def _():
        sup_ref[...] = jnp.tanh(x_ref[...] @ w1_ref[...]).astype(jnp.bfloat16)

    a = adj_ref[...].astype(jnp.bfloat16)
    rows = pl.ds(i * _BMS, _BMS)
    adj16_ref[rows, :] = a
    feat_ref[rows, :] = jax.lax.dot_general(
        a, sup_ref[...], (((1,), (0,)), ((), ())),
        preferred_element_type=jnp.float32)

    @pl.when(i == _NBS - 1)
    def _():
        def spmm(sup, dst_ref):
            def body(j, _):
                r = pl.ds(j * _BMR, _BMR)
                dst_ref[r, :] = jax.lax.dot_general(
                    adj16_ref[r, :], sup,
                    (((1,), (0,)), ((), ())),
                    preferred_element_type=jnp.float32)
                return 0

            jax.lax.fori_loop(0, _NBR, body, 0)

        def support(src_ref, w_ref, active):
            s = src_ref[...] @ w_ref[...]
            if active:
                s = jnp.tanh(s)
            return s.astype(jnp.bfloat16)

        slots = (slot0_ref, slot1_ref)
        pend = [None, None]

        def gram_blocks(out_hbm_ref):
            for j in range(_NBG):
                k = j % 2
                if pend[k] is not None:
                    pend[k].wait()
                s = jax.lax.dot_general(
                    zb_ref[j * _BMG:(j + 1) * _BMG, :], zb_ref[...],
                    (((1,), (1,)), ((), ())),
                    preferred_element_type=jnp.float32)
                slots[k][...] = jax.nn.sigmoid(s)
                dma = pltpu.make_async_copy(
                    slots[k], out_hbm_ref.at[pl.ds(j * _BMG, _BMG), :],
                    sem_ref.at[k])
                dma.start()
                pend[k] = dma

        spmm(support(feat_ref, w2_ref, True), feat_ref)
        spmm(support(feat_ref, w3_ref, False), feat_ref)
        zig_ref[...] = feat_ref[:, :32]
        zb_ref[...] = feat_ref[...].astype(jnp.bfloat16)
        sup4 = support(feat_ref, w4_ref, True)
        gram_blocks(g1_ref)
        spmm(sup4, feat_ref)
        spmm(support(feat_ref, w5_ref, True), feat_ref)
        spmm(support(feat_ref, w6_ref, True), zhat_ref)
        zb_ref[...] = zhat_ref[...].astype(jnp.bfloat16)
        gram_blocks(g2_ref)
        pend[0].wait()
        pend[1].wait()


def _pad_w(w):
    fin, fout = w.shape
    return jnp.pad(w, ((0, _F - fin), (0, _F - fout)))


def kernel(x, adj, W1, W2, W3, W4, W5, W6):
    ws = [_pad_w(w) for w in (W1, W2, W3, W4, W5, W6)]
    z_igae, z_igae_adj, z_hat, z_hat_adj = pl.pallas_call(
        _model_kernel,
        grid=(_NBS,),
        in_specs=[
            pl.BlockSpec((_BMS, _N), lambda i: (i, 0)),
            pl.BlockSpec((_N, _F), lambda i: (0, 0)),
        ] + [pl.BlockSpec((_F, _F), lambda i: (0, 0))] * 6,
        out_specs=[
            pl.BlockSpec((_N, 32), lambda i: (0, 0)),
            pl.BlockSpec(memory_space=pl.ANY),
            pl.BlockSpec((_N, _F), lambda i: (0, 0)),
            pl.BlockSpec(memory_space=pl.ANY),
        ],
        out_shape=[
            jax.ShapeDtypeStruct((_N, 32), jnp.float32),
            jax.ShapeDtypeStruct((_N, _N), jnp.float32),
            jax.ShapeDtypeStruct((_N, _F), jnp.float32),
            jax.ShapeDtypeStruct((_N, _N), jnp.float32),
        ],
        scratch_shapes=[
            pltpu.VMEM((_N, _N), jnp.bfloat16),
            pltpu.VMEM((_N, _F), jnp.float32),
            pltpu.VMEM((_N, _F), jnp.bfloat16),
            pltpu.VMEM((_N, _F), jnp.bfloat16),
            pltpu.VMEM((_BMG, _N), jnp.float32),
            pltpu.VMEM((_BMG, _N), jnp.float32),
            pltpu.SemaphoreType.DMA((2,)),
        ],
    )(adj, x, *ws)
    return (z_igae, z_igae_adj, z_hat, z_hat_adj)


# speedup vs baseline: 1.2222x; 1.2222x over previous
import jax
import jax.numpy as jnp
from jax.experimental import pallas as pl
from jax.experimental.pallas import tpu as pltpu


_N = 4096
_F = 128
_BMS = 256
_NBS = _N // _BMS
_BMR = 512
_NBR = _N // _BMR


def _encdec_kernel(adj_ref, x_ref, w1_ref, w2_ref, w3_ref, w4_ref, w5_ref,
                   w6_ref, zig_ref, zigp_ref, zhat_ref,
                   adj16_ref, feat_ref, sup_ref):
    i = pl.program_id(0)

    @pl.when(i == 0)
    def _():
        sup_ref[...] = jnp.tanh(x_ref[...] @ w1_ref[...]).astype(jnp.bfloat16)

    a = adj_ref[...].astype(jnp.bfloat16)
    rows = pl.ds(i * _BMS, _BMS)
    adj16_ref[rows, :] = a
    feat_ref[rows, :] = jax.lax.dot_general(
        a, sup_ref[...], (((1,), (0,)), ((), ())),
        preferred_element_type=jnp.float32)

    @pl.when(i == _NBS - 1)
    def _():
        def spmm(sup, dst_ref):
            def body(j, _):
                r = pl.ds(j * _BMR, _BMR)
                dst_ref[r, :] = jax.lax.dot_general(
                    adj16_ref[r, :], sup,
                    (((1,), (0,)), ((), ())),
                    preferred_element_type=jnp.float32)
                return 0

            jax.lax.fori_loop(0, _NBR, body, 0)

        def support(src_ref, w_ref, active):
            s = src_ref[...] @ w_ref[...]
            if active:
                s = jnp.tanh(s)
            return s.astype(jnp.bfloat16)

        spmm(support(feat_ref, w2_ref, True), feat_ref)
        spmm(support(feat_ref, w3_ref, False), zigp_ref)
        zig_ref[...] = zigp_ref[:, :32]
        spmm(support(zigp_ref, w4_ref, True), feat_ref)
        spmm(support(feat_ref, w5_ref, True), feat_ref)
        spmm(support(feat_ref, w6_ref, True), zhat_ref)


def _gram_kernel(z_ref, zfull_ref, out_ref, zf_ref):
    @pl.when(pl.program_id(0) == 0)
    def _():
        zf_ref[...] = zfull_ref[...].astype(jnp.bfloat16)

    zb = z_ref[...].astype(jnp.bfloat16)
    s = jax.lax.dot_general(
        zb, zf_ref[...], (((1,), (1,)), ((), ())),
        preferred_element_type=jnp.float32)
    out_ref[...] = jax.nn.sigmoid(s)


def _gram(z, block_rows=1024):
    n, f = z.shape
    return pl.pallas_call(
        _gram_kernel,
        grid=(n // block_rows,),
        in_specs=[
            pl.BlockSpec((block_rows, f), lambda i: (i, 0)),
            pl.BlockSpec((n, f), lambda i: (0, 0)),
        ],
        out_specs=pl.BlockSpec((block_rows, n), lambda i: (i, 0)),
        out_shape=jax.ShapeDtypeStruct((n, n), jnp.float32),
        scratch_shapes=[pltpu.VMEM((n, f), jnp.bfloat16)],
    )(z, z)


def _pad_w(w):
    fin, fout = w.shape
    return jnp.pad(w, ((0, _F - fin), (0, _F - fout)))


def kernel(x, adj, W1, W2, W3, W4, W5, W6):
    ws = [_pad_w(w) for w in (W1, W2, W3, W4, W5, W6)]
    z_igae, zig_pad, z_hat = pl.pallas_call(
        _encdec_kernel,
        grid=(_NBS,),
        in_specs=[
            pl.BlockSpec((_BMS, _N), lambda i: (i, 0)),
            pl.BlockSpec((_N, _F), lambda i: (0, 0)),
        ] + [pl.BlockSpec((_F, _F), lambda i: (0, 0))] * 6,
        out_specs=[
            pl.BlockSpec((_N, 32), lambda i: (0, 0)),
            pl.BlockSpec((_N, _F), lambda i: (0, 0)),
            pl.BlockSpec((_N, _F), lambda i: (0, 0)),
        ],
        out_shape=[
            jax.ShapeDtypeStruct((_N, 32), jnp.float32),
            jax.ShapeDtypeStruct((_N, _F), jnp.float32),
            jax.ShapeDtypeStruct((_N, _F), jnp.float32),
        ],
        scratch_shapes=[
            pltpu.VMEM((_N, _N), jnp.bfloat16),
            pltpu.VMEM((_N, _F), jnp.float32),
            pltpu.VMEM((_N, _F), jnp.bfloat16),
        ],
    )(adj, x, *ws)
    z_igae_adj = _gram(zig_pad)
    z_hat_adj = _gram(z_hat)
    return (z_igae, z_igae_adj, z_hat, z_hat_adj)


# speedup vs baseline: 1.2250x; 1.0023x over previous
import jax
import jax.numpy as jnp
from jax.experimental import pallas as pl
from jax.experimental.pallas import tpu as pltpu


_N = 4096
_F = 128
_BMS = 256
_NBS = _N // _BMS
_BMR = 512
_NBR = _N // _BMR


def _encdec_kernel(adj_ref, x_ref, w1_ref, w2_ref, w3_ref, w4_ref, w5_ref,
                   w6_ref, zig_ref, zigp_ref, zhat_ref,
                   adj16_ref, feat_ref, sup_ref):
    i = pl.program_id(0)

    @pl.when(i == 0)
    def _():
        sup_ref[...] = jnp.tanh(x_ref[...] @ w1_ref[...]).astype(jnp.bfloat16)

    a = adj_ref[...].astype(jnp.bfloat16)
    rows = pl.ds(i * _BMS, _BMS)
    adj16_ref[rows, :] = a
    feat_ref[rows, :] = jax.lax.dot_general(
        a, sup_ref[...], (((1,), (0,)), ((), ())),
        preferred_element_type=jnp.float32)

    @pl.when(i == _NBS - 1)
    def _():
        def spmm(sup, dst_ref):
            for j in range(_NBR):
                r = pl.ds(j * _BMR, _BMR)
                dst_ref[r, :] = jax.lax.dot_general(
                    adj16_ref[r, :], sup,
                    (((1,), (0,)), ((), ())),
                    preferred_element_type=jnp.float32)

        def support(src_ref, w_ref, active):
            s = src_ref[...] @ w_ref[...]
            if active:
                s = jnp.tanh(s)
            return s.astype(jnp.bfloat16)

        spmm(support(feat_ref, w2_ref, True), feat_ref)
        spmm(support(feat_ref, w3_ref, False), zigp_ref)
        zig_ref[...] = zigp_ref[:, :32]
        spmm(support(zigp_ref, w4_ref, True), feat_ref)
        spmm(support(feat_ref, w5_ref, True), feat_ref)
        spmm(support(feat_ref, w6_ref, True), zhat_ref)


def _gram_kernel(z_ref, zfull_ref, out_ref, zf_ref):
    @pl.when(pl.program_id(0) == 0)
    def _():
        zf_ref[...] = zfull_ref[...].astype(jnp.bfloat16)

    zb = z_ref[...].astype(jnp.bfloat16)
    s = jax.lax.dot_general(
        zb, zf_ref[...], (((1,), (1,)), ((), ())),
        preferred_element_type=jnp.float32)
    out_ref[...] = jax.nn.sigmoid(s)


def _gram(z, block_rows=1024):
    n, f = z.shape
    return pl.pallas_call(
        _gram_kernel,
        grid=(n // block_rows,),
        in_specs=[
            pl.BlockSpec((block_rows, f), lambda i: (i, 0)),
            pl.BlockSpec((n, f), lambda i: (0, 0)),
        ],
        out_specs=pl.BlockSpec((block_rows, n), lambda i: (i, 0)),
        out_shape=jax.ShapeDtypeStruct((n, n), jnp.float32),
        scratch_shapes=[pltpu.VMEM((n, f), jnp.bfloat16)],
    )(z, z)


def _pad_w(w):
    fin, fout = w.shape
    return jnp.pad(w, ((0, _F - fin), (0, _F - fout)))


def kernel(x, adj, W1, W2, W3, W4, W5, W6):
    ws = [_pad_w(w) for w in (W1, W2, W3, W4, W5, W6)]
    z_igae, zig_pad, z_hat = pl.pallas_call(
        _encdec_kernel,
        grid=(_NBS,),
        in_specs=[
            pl.BlockSpec((_BMS, _N), lambda i: (i, 0)),
            pl.BlockSpec((_N, _F), lambda i: (0, 0)),
        ] + [pl.BlockSpec((_F, _F), lambda i: (0, 0))] * 6,
        out_specs=[
            pl.BlockSpec((_N, 32), lambda i: (0, 0)),
            pl.BlockSpec((_N, _F), lambda i: (0, 0)),
            pl.BlockSpec((_N, _F), lambda i: (0, 0)),
        ],
        out_shape=[
            jax.ShapeDtypeStruct((_N, 32), jnp.float32),
            jax.ShapeDtypeStruct((_N, _F), jnp.float32),
            jax.ShapeDtypeStruct((_N, _F), jnp.float32),
        ],
        scratch_shapes=[
            pltpu.VMEM((_N, _N), jnp.bfloat16),
            pltpu.VMEM((_N, _F), jnp.float32),
            pltpu.VMEM((_N, _F), jnp.bfloat16),
        ],
    )(adj, x, *ws)
    z_igae_adj = _gram(zig_pad)
    z_hat_adj = _gram(z_hat)
    return (z_igae, z_igae_adj, z_hat, z_hat_adj)
